# baseline (speedup 1.0000x reference)
import functools

import jax
import jax.numpy as jnp
from jax import lax
from jax.experimental import pallas as pl
from jax.experimental.pallas import tpu as pltpu
from jax.experimental.pallas import tpu_sc as plsc

_NC = 2
_NS = 16
_NW = _NC * _NS


def _pack16(lo, hi):
    ul = jax.lax.bitcast_convert_type(lo, jnp.uint32)
    uh = jax.lax.bitcast_convert_type(hi, jnp.uint32)
    half = jnp.uint32(0x8000)
    w = ((ul + half) >> 16) | ((uh + half) & jnp.uint32(0xFFFF0000))
    return jax.lax.bitcast_convert_type(w, jnp.float32)


def _sc_gather_combine(xt, idx_flat, Es, C, K):
    e_per_w = Es // _NW
    nchunks = e_per_w // K
    assert nchunks % 2 == 1 and K % 8 == 0 and e_per_w % K == 0
    mesh = plsc.VectorSubcoreMesh(core_axis_name="c", subcore_axis_name="s")

    row_t = pltpu.VMEM((4 * K, C), jnp.float32)
    out_t = pltpu.VMEM((K, C), jnp.float32)

    @functools.partial(
        pl.kernel,
        mesh=mesh,
        out_type=jax.ShapeDtypeStruct((2 * Es, C), jnp.float32),
        scratch_types=[
            pltpu.VMEM((4 * e_per_w,), jnp.int32),
            row_t, row_t,
            out_t, out_t,
            out_t, out_t,
            pltpu.SemaphoreType.DMA,
            pltpu.SemaphoreType.DMA,
            pltpu.SemaphoreType.DMA,
            pltpu.SemaphoreType.DMA,
        ],
    )
    def gather_kernel(xt_hbm, idx_hbm, out_hbm,
                      iw, ra, rb, oa0, oa1, ob0, ob1,
                      sga, sgb, ssa, ssb):
        wid = lax.axis_index("s") * _NC + lax.axis_index("c")
        w_base = wid * e_per_w
        rows = (ra, rb)
        outs = ((oa0, oa1), (ob0, ob1))
        sg = (sga, sgb)
        ss = (ssa, ssb)

        pltpu.sync_copy(idx_hbm.at[pl.ds(4 * w_base, 4 * e_per_w)], iw)

        def fire_g(c, p):
            pltpu.async_copy(
                xt_hbm.at[iw.at[pl.ds(c * 4 * K, 4 * K)]], rows[p], sg[p])

        def fire_s(c, p):
            base = pl.multiple_of(w_base + c * K, 8)
            for h in range(2):
                pltpu.async_copy(outs[p][h],
                                 out_hbm.at[pl.ds(h * Es + base, K)],
                                 ss[p])

        def wait_g(p):
            pltpu.make_async_copy(xt_hbm.at[pl.ds(0, 4 * K)], rows[p],
                                  sg[p]).wait()

        def wait_s(p):
            for h in range(2):
                pltpu.make_async_copy(xt_hbm.at[pl.ds(0, K)], outs[p][h],
                                      ss[p]).wait()

        def compute(p):
            r = rows[p]
            o1, o2 = outs[p]

            def edge_body(e, carry):
                for ja, jb, o in ((0, 2, o1), (1, 3, o2)):
                    for k in range(4):
                        alo = r[4 * e + ja, pl.ds(16 * k, 16)]
                        blo = r[4 * e + jb, pl.ds(16 * k, 16)]
                        ahi = r[4 * e + ja, pl.ds(64 + 16 * k, 16)]
                        bhi = r[4 * e + jb, pl.ds(64 + 16 * k, 16)]
                        o[e, pl.ds(16 * k, 16)] = _pack16(
                            alo + blo, ahi + bhi)
                        o[e, pl.ds(64 + 16 * k, 16)] = _pack16(
                            jnp.abs(alo - blo), jnp.abs(ahi - bhi))
                return carry

            lax.fori_loop(0, K, edge_body, 0)

        fire_g(0, 0)

        def body(u, carry):
            ca = 2 * u
            wait_g(0)

            @pl.when(ca + 1 < nchunks)
            def _():
                fire_g(ca + 1, 1)

            @pl.when(u > 0)
            def _():
                wait_s(0)

            compute(0)
            fire_s(ca, 0)

            @pl.when(ca + 1 < nchunks)
            def _():
                wait_g(1)

                @pl.when(ca + 2 < nchunks)
                def _():
                    fire_g(ca + 2, 0)

                @pl.when(u > 0)
                def _():
                    wait_s(1)

                compute(1)
                fire_s(ca + 1, 1)

            return carry

        lax.fori_loop(0, (nchunks + 1) // 2, body, 0)
        wait_s(0)
        wait_s(1)

    return gather_kernel(xt, idx_flat)


def _unpack_words(w):
    wi = jax.lax.bitcast_convert_type(w, jnp.uint32)
    lo = jax.lax.bitcast_convert_type(wi << 16, jnp.float32)
    hi = jax.lax.bitcast_convert_type(
        wi & jnp.uint32(0xFFFF0000), jnp.float32)
    return lo, hi


def _tc_combine_conv(xt, comb, wcat, bias, Es, C, Eb):
    def body(xt_ref, comb_ref, w_ref, b_ref, out_ref):
        f0 = xt_ref[...]
        c1l, c1h = _unpack_words(comb_ref[0])
        c2l, c2h = _unpack_words(comb_ref[1])
        feat = jnp.concatenate(
            [f0, c1l, c1h, c2l, c2h], axis=-1)
        acc = lax.dot_general(
            w_ref[...], feat, (((0,), (1,)), ((), ())),
            preferred_element_type=jnp.float32)
        out_ref[...] = acc + b_ref[...]

    return pl.pallas_call(
        body,
        grid=(Es // Eb,),
        in_specs=[
            pl.BlockSpec((Eb, C), lambda i: (i, 0)),
            pl.BlockSpec((2, Eb, C), lambda i: (0, i, 0)),
            pl.BlockSpec((5 * C, 32), lambda i: (0, 0)),
            pl.BlockSpec((32, 1), lambda i: (0, 0)),
        ],
        out_specs=pl.BlockSpec((32, Eb), lambda i: (0, i)),
        out_shape=jax.ShapeDtypeStruct((32, Es), jnp.float32),
    )(xt, comb, wcat, bias)


def kernel(x, gemm_edges, W, b):
    Bq, C, E = x.shape
    xt = jnp.transpose(x[0])
    idx_flat = gemm_edges[0].reshape(-1)

    comb = _sc_gather_combine(xt, idx_flat, E, C, K=40)
    comb = comb.reshape(2, E, C)

    w5 = jnp.transpose(W[:, :, 0, :], (2, 1, 0))
    h = C // 2
    wcat = jnp.concatenate(
        [w5[0],
         w5[1][:h], w5[3][:h], w5[1][h:], w5[3][h:],
         w5[2][:h], w5[4][:h], w5[2][h:], w5[4][h:]], axis=0)
    out = _tc_combine_conv(xt, comb, wcat, b.reshape(32, 1), E, C, Eb=1280)
    return out[None, :, :, None]


# speedup vs baseline: 1.4962x; 1.4962x over previous
import functools

import jax
import jax.numpy as jnp
from jax import lax
from jax.experimental import pallas as pl
from jax.experimental.pallas import tpu as pltpu
from jax.experimental.pallas import tpu_sc as plsc

_NC = 2
_NS = 16
_NW = _NC * _NS


def _pack16(lo, hi):
    ul = jax.lax.bitcast_convert_type(lo, jnp.uint32)
    uh = jax.lax.bitcast_convert_type(hi, jnp.uint32)
    half = jnp.uint32(0x8000)
    w = ((ul + half) >> 16) | ((uh + half) & jnp.uint32(0xFFFF0000))
    return jax.lax.bitcast_convert_type(w, jnp.float32)


def _sc_gather_combine(xt, idx_flat, Es, C, K):
    e_per_w = Es // _NW
    nchunks = e_per_w // K
    assert nchunks % 2 == 1 and K % 8 == 0 and e_per_w % K == 0
    mesh = plsc.VectorSubcoreMesh(core_axis_name="c", subcore_axis_name="s")

    row_t = pltpu.VMEM((K, C), jnp.float32)

    @functools.partial(
        pl.kernel,
        mesh=mesh,
        out_type=jax.ShapeDtypeStruct((2 * Es, C), jnp.float32),
        scratch_types=[
            pltpu.VMEM((e_per_w,), jnp.int32),
            pltpu.VMEM((e_per_w,), jnp.int32),
            pltpu.VMEM((e_per_w,), jnp.int32),
            pltpu.VMEM((e_per_w,), jnp.int32),
            row_t, row_t, row_t, row_t,
            row_t, row_t, row_t, row_t,
            row_t, row_t,
            row_t, row_t,
            pltpu.SemaphoreType.DMA,
            pltpu.SemaphoreType.DMA,
            pltpu.SemaphoreType.DMA,
            pltpu.SemaphoreType.DMA,
        ],
    )
    def gather_kernel(xt_hbm, idx_hbm, out_hbm,
                      i0, i1, i2, i3,
                      a0, a1, a2, a3, b0, b1, b2, b3,
                      oa0, oa1, ob0, ob1,
                      sga, sgb, ssa, ssb):
        wid = lax.axis_index("s") * _NC + lax.axis_index("c")
        w_base = wid * e_per_w
        idxw = (i0, i1, i2, i3)
        rows = ((a0, a1, a2, a3), (b0, b1, b2, b3))
        outs = ((oa0, oa1), (ob0, ob1))
        sg = (sga, sgb)
        ss = (ssa, ssb)

        for j in range(4):
            pltpu.sync_copy(idx_hbm.at[pl.ds(j * Es + w_base, e_per_w)],
                            idxw[j])

        def fire_g(c, p):
            for j in range(4):
                pltpu.async_copy(
                    xt_hbm.at[idxw[j].at[pl.ds(c * K, K)]], rows[p][j],
                    sg[p])

        def fire_s(c, p):
            base = pl.multiple_of(w_base + c * K, 8)
            for h in range(2):
                pltpu.async_copy(outs[p][h],
                                 out_hbm.at[pl.ds(h * Es + base, K)],
                                 ss[p])

        def wait_g(p):
            for j in range(4):
                pltpu.make_async_copy(xt_hbm.at[pl.ds(0, K)], rows[p][j],
                                      sg[p]).wait()

        def wait_s(p):
            for h in range(2):
                pltpu.make_async_copy(xt_hbm.at[pl.ds(0, K)], outs[p][h],
                                      ss[p]).wait()

        def compute(p):
            r1, r2, r3, r4 = rows[p]
            o1, o2 = outs[p]

            def edge_body(e, carry):
                for ra, rb, o in ((r1, r3, o1), (r2, r4, o2)):
                    for k in range(4):
                        alo = ra[e, pl.ds(16 * k, 16)]
                        blo = rb[e, pl.ds(16 * k, 16)]
                        ahi = ra[e, pl.ds(64 + 16 * k, 16)]
                        bhi = rb[e, pl.ds(64 + 16 * k, 16)]
                        o[e, pl.ds(16 * k, 16)] = _pack16(
                            alo + blo, ahi + bhi)
                        o[e, pl.ds(64 + 16 * k, 16)] = _pack16(
                            jnp.abs(alo - blo), jnp.abs(ahi - bhi))
                return carry

            lax.fori_loop(0, K, edge_body, 0)

        fire_g(0, 0)

        def body(u, carry):
            ca = 2 * u
            wait_g(0)

            @pl.when(ca + 1 < nchunks)
            def _():
                fire_g(ca + 1, 1)

            @pl.when(u > 0)
            def _():
                wait_s(0)

            compute(0)
            fire_s(ca, 0)

            @pl.when(ca + 1 < nchunks)
            def _():
                wait_g(1)

                @pl.when(ca + 2 < nchunks)
                def _():
                    fire_g(ca + 2, 0)

                @pl.when(u > 0)
                def _():
                    wait_s(1)

                compute(1)
                fire_s(ca + 1, 1)

            return carry

        lax.fori_loop(0, (nchunks + 1) // 2, body, 0)
        wait_s(0)
        wait_s(1)

    return gather_kernel(xt, idx_flat)


def _unpack_words(w):
    wi = jax.lax.bitcast_convert_type(w, jnp.uint32)
    lo = jax.lax.bitcast_convert_type(wi << 16, jnp.float32)
    hi = jax.lax.bitcast_convert_type(
        wi & jnp.uint32(0xFFFF0000), jnp.float32)
    return lo, hi


def _tc_combine_conv(xt, comb, wcat, bias, Es, C, Eb):
    def body(xt_ref, comb_ref, w_ref, b_ref, out_ref):
        f0 = xt_ref[...]
        c1l, c1h = _unpack_words(comb_ref[0])
        c2l, c2h = _unpack_words(comb_ref[1])
        feat = jnp.concatenate(
            [f0, c1l, c1h, c2l, c2h], axis=-1)
        acc = lax.dot_general(
            w_ref[...], feat, (((0,), (1,)), ((), ())),
            preferred_element_type=jnp.float32)
        out_ref[...] = acc + b_ref[...]

    return pl.pallas_call(
        body,
        grid=(Es // Eb,),
        in_specs=[
            pl.BlockSpec((Eb, C), lambda i: (i, 0)),
            pl.BlockSpec((2, Eb, C), lambda i: (0, i, 0)),
            pl.BlockSpec((5 * C, 32), lambda i: (0, 0)),
            pl.BlockSpec((32, 1), lambda i: (0, 0)),
        ],
        out_specs=pl.BlockSpec((32, Eb), lambda i: (0, i)),
        out_shape=jax.ShapeDtypeStruct((32, Es), jnp.float32),
    )(xt, comb, wcat, bias)


def kernel(x, gemm_edges, W, b):
    Bq, C, E = x.shape
    xt = jnp.transpose(x[0])
    idx_flat = jnp.transpose(gemm_edges[0]).reshape(-1)

    comb = _sc_gather_combine(xt, idx_flat, E, C, K=40)
    comb = comb.reshape(2, E, C)

    w5 = jnp.transpose(W[:, :, 0, :], (2, 1, 0))
    h = C // 2
    wcat = jnp.concatenate(
        [w5[0],
         w5[1][:h], w5[3][:h], w5[1][h:], w5[3][h:],
         w5[2][:h], w5[4][:h], w5[2][h:], w5[4][h:]], axis=0)
    out = _tc_combine_conv(xt, comb, wcat, b.reshape(32, 1), E, C, Eb=1280)
    return out[None, :, :, None]
